# baseline (speedup 1.0000x reference)
import jax
import jax.numpy as jnp
from jax.experimental import pallas as pl
from jax.experimental.pallas import tpu as pltpu
from jax.experimental.pallas import tpu_sc as plsc

_EMB = 64
_SCALE = 8.0
_W = 128


def kernel(x, table):
    B, S = x.shape
    n = B * S
    idx = x.reshape(1, n)
    mesh = plsc.VectorSubcoreMesh(core_axis_name="c", subcore_axis_name="s")

    @pl.kernel(
        out_type=jax.ShapeDtypeStruct((n, _EMB), table.dtype),
        mesh=mesh,
        compiler_params=pltpu.CompilerParams(use_tc_tiling_on_sc=False),
    )
    def _gather(tab_hbm, i_hbm, o_hbm):
        def body(i_vmem, o_vmem):
            pltpu.sync_copy(tab_hbm.at[i_vmem.at[0]], o_vmem)

            @pl.loop(0, _W)
            def _row(r):
                for c in range(0, _EMB, 16):
                    o_vmem.at[r, pl.ds(c, 16)][...] = (
                        o_vmem.at[r, pl.ds(c, 16)][...] * _SCALE
                    )

        pltpu.emit_pipeline(
            body,
            grid=(n // _W,),
            in_specs=[pl.BlockSpec((1, _W), index_map=lambda i: (0, i))],
            out_specs=[pl.BlockSpec((_W, _EMB), index_map=lambda i: (i, 0))],
            core_axis_name=("c", "s"),
            dimension_semantics=(pltpu.PARALLEL,),
        )(i_hbm, o_hbm)

    out = _gather(table, idx)
    return out.reshape(B, S, _EMB)


# speedup vs baseline: 1.4947x; 1.4947x over previous
import jax
import jax.numpy as jnp
from jax.experimental import pallas as pl
from jax.experimental.pallas import tpu as pltpu
from jax.experimental.pallas import tpu_sc as plsc

_EMB = 64
_SCALE = 8.0
_W = 128
_NBUF = 4
_NC, _NS = 2, 16
_NWORK = _NC * _NS


def kernel(x, table):
    B, S = x.shape
    n = B * S
    n_win = n // (_W * _NWORK)
    idx = x.reshape(n // _W, _W)
    mesh = plsc.VectorSubcoreMesh(core_axis_name="c", subcore_axis_name="s")

    @pl.kernel(
        out_type=jax.ShapeDtypeStruct((n, _EMB), table.dtype),
        mesh=mesh,
        compiler_params=pltpu.CompilerParams(use_tc_tiling_on_sc=False),
        scratch_types=(
            [pltpu.VMEM((n_win, _W), jnp.int32)]
            + [pltpu.VMEM((_W, _EMB), jnp.float32) for _ in range(2 * _NBUF)]
            + [pltpu.SemaphoreType.DMA for _ in range(2 * _NBUF + 1)]
        ),
    )
    def _gather(tab_hbm, i_hbm, o_hbm, idx_v, *bufs_and_sems):
        gbuf = bufs_and_sems[:_NBUF]
        obuf = bufs_and_sems[_NBUF : 2 * _NBUF]
        gsem = bufs_and_sems[2 * _NBUF : 3 * _NBUF]
        osem = bufs_and_sems[3 * _NBUF : 4 * _NBUF]
        isem = bufs_and_sems[4 * _NBUF]

        wid = jax.lax.axis_index("s") * _NC + jax.lax.axis_index("c")
        win0 = wid * n_win

        pltpu.async_copy(i_hbm.at[pl.ds(win0, n_win)], idx_v, isem).wait()

        def start_gather(b, g):
            pltpu.make_async_copy(
                tab_hbm.at[idx_v.at[g]], gbuf[b], gsem[b]
            ).start()

        def start_out(b, g):
            pltpu.make_async_copy(
                obuf[b], o_hbm.at[pl.ds((win0 + g) * _W, _W)], osem[b]
            ).start()

        for b in range(_NBUF):
            start_gather(b, b)

        @pl.loop(0, n_win, step=_NBUF)
        def _round(t):
            for b in range(_NBUF):
                g = t + b
                pltpu.make_async_copy(
                    tab_hbm.at[idx_v.at[g]], gbuf[b], gsem[b]
                ).wait()

                @pl.when(t > 0)
                def _():
                    pltpu.make_async_copy(
                        obuf[b], o_hbm.at[pl.ds((win0 + g) * _W, _W)], osem[b]
                    ).wait()

                @pl.loop(0, _W)
                def _row(r):
                    for c in range(0, _EMB, 16):
                        obuf[b].at[r, pl.ds(c, 16)][...] = (
                            gbuf[b].at[r, pl.ds(c, 16)][...] * _SCALE
                        )

                @pl.when(g + _NBUF < n_win)
                def _():
                    start_gather(b, g + _NBUF)

                start_out(b, g)

        for b in range(_NBUF):
            pltpu.make_async_copy(
                obuf[b],
                o_hbm.at[pl.ds((win0 + n_win - _NBUF + b) * _W, _W)],
                osem[b],
            ).wait()

    out = _gather(table, idx)
    return out.reshape(B, S, _EMB)
